# baseline (speedup 1.0000x reference)
import functools

import jax
import jax.numpy as jnp
from jax.experimental import pallas as pl
from jax.experimental.pallas import tpu as pltpu

B, T, I, H, V, D = 8, 256, 1024, 8, 10000, 512

TT = 128
VT = 2048
NV = (V + VT - 1) // VT


def _head_kernel(ah_ref, enc_ref, dec_ref, tar_ref, w_ref, b_ref,
                 attn_ref, pgen_ref, loss_ref):
    bi = pl.program_id(0)
    tj = pl.program_id(1)
    attn = jnp.mean(ah_ref[0], axis=0)
    attn_ref[0] = attn
    ctx = jnp.dot(attn, enc_ref[0], preferred_element_type=jnp.float32)
    cat = jnp.concatenate([ctx, dec_ref[0], tar_ref[0]], axis=1)
    logits = jnp.dot(cat, w_ref[...], preferred_element_type=jnp.float32)
    logits = logits + b_ref[0, 0]
    pg = jax.nn.sigmoid(logits)
    pgen_ref[...] = pg.reshape(1, 1, TT)
    partial = jnp.sum(10.0 * jax.nn.relu(jnp.abs(pg - 0.5) - 0.45))

    @pl.when(jnp.logical_and(bi == 0, tj == 0))
    def _():
        loss_ref[...] = jnp.zeros((1, 1), jnp.float32)

    loss_ref[...] += partial.reshape(1, 1) / (B * T)


def _run_head(attn_heads, enc, dec, tar, w, bvec):
    grid = (B, T // TT)
    return pl.pallas_call(
        _head_kernel,
        grid=grid,
        in_specs=[
            pl.BlockSpec((1, H, TT, I), lambda b, t: (b, 0, t, 0)),
            pl.BlockSpec((1, I, D), lambda b, t: (b, 0, 0)),
            pl.BlockSpec((1, TT, D), lambda b, t: (b, t, 0)),
            pl.BlockSpec((1, TT, D), lambda b, t: (b, t, 0)),
            pl.BlockSpec((3 * D, 1), lambda b, t: (0, 0)),
            pl.BlockSpec((1, 1), lambda b, t: (0, 0)),
        ],
        out_specs=[
            pl.BlockSpec((1, TT, I), lambda b, t: (b, t, 0)),
            pl.BlockSpec((1, 1, TT), lambda b, t: (b, 0, t)),
            pl.BlockSpec((1, 1), lambda b, t: (0, 0)),
        ],
        out_shape=[
            jax.ShapeDtypeStruct((B, T, I), jnp.float32),
            jax.ShapeDtypeStruct((B, 1, T), jnp.float32),
            jax.ShapeDtypeStruct((1, 1), jnp.float32),
        ],
    )(attn_heads, enc, dec, tar, w, bvec.reshape(1, 1))


def _stats_kernel(s_ref, m_ref, z_ref, ms_ref, zs_ref):
    vj = pl.program_id(1)

    @pl.when(vj == 0)
    def _():
        ms_ref[...] = jnp.full((T, 1), -1e30, jnp.float32)
        zs_ref[...] = jnp.zeros((T, 1), jnp.float32)

    v_base = vj * VT
    col = jax.lax.broadcasted_iota(jnp.int32, (T, VT), 1)
    valid = (col + v_base) < V
    s = jnp.where(valid, s_ref[0], -1e30)
    tile_max = jnp.max(s, axis=1, keepdims=True)
    m_old = ms_ref[...]
    m_new = jnp.maximum(m_old, tile_max)
    e = jnp.where(valid, jnp.exp(s - m_new), 0.0)
    zs_ref[...] = zs_ref[...] * jnp.exp(m_old - m_new) + jnp.sum(
        e, axis=1, keepdims=True)
    ms_ref[...] = m_new

    @pl.when(vj == NV - 1)
    def _():
        m_ref[...] = ms_ref[...].reshape(1, 1, T)
        z_ref[...] = zs_ref[...].reshape(1, 1, T)


def _run_stats(s):
    return pl.pallas_call(
        _stats_kernel,
        grid=(B, NV),
        in_specs=[pl.BlockSpec((1, T, VT), lambda b, v: (b, 0, v))],
        out_specs=[
            pl.BlockSpec((1, 1, T), lambda b, v: (b, 0, 0)),
            pl.BlockSpec((1, 1, T), lambda b, v: (b, 0, 0)),
        ],
        out_shape=[
            jax.ShapeDtypeStruct((B, 1, T), jnp.float32),
            jax.ShapeDtypeStruct((B, 1, T), jnp.float32),
        ],
        scratch_shapes=[
            pltpu.VMEM((T, 1), jnp.float32),
            pltpu.VMEM((T, 1), jnp.float32),
        ],
    )(s)


def _mix_kernel(s_ref, gen_ref, pg_ref, m_ref, z_ref, ptr_ref, fin_ref):
    m = m_ref[0, 0].reshape(T, 1)
    zinv = 1.0 / z_ref[0, 0].reshape(T, 1)
    pg = pg_ref[0, 0].reshape(T, 1)
    ptr = jnp.exp(s_ref[0] - m) * zinv
    ptr_ref[0] = ptr
    fin_ref[0] = pg * gen_ref[0] + (1.0 - pg) * ptr


def _run_mix(s, gen, pg, m, z):
    return pl.pallas_call(
        _mix_kernel,
        grid=(B, NV),
        in_specs=[
            pl.BlockSpec((1, T, VT), lambda b, v: (b, 0, v)),
            pl.BlockSpec((1, T, VT), lambda b, v: (b, 0, v)),
            pl.BlockSpec((1, 1, T), lambda b, v: (b, 0, 0)),
            pl.BlockSpec((1, 1, T), lambda b, v: (b, 0, 0)),
            pl.BlockSpec((1, 1, T), lambda b, v: (b, 0, 0)),
        ],
        out_specs=[
            pl.BlockSpec((1, T, VT), lambda b, v: (b, 0, v)),
            pl.BlockSpec((1, T, VT), lambda b, v: (b, 0, v)),
        ],
        out_shape=[
            jax.ShapeDtypeStruct((B, T, V), jnp.float32),
            jax.ShapeDtypeStruct((B, T, V), jnp.float32),
        ],
    )(s, gen, pg, m, z)


def kernel(inp_tokens, tar_embedded, generator_output, enc_output, dec_state,
           attn_heads, W_pgen, b_pgen):
    attn, p_gen3, loss = _run_head(attn_heads, enc_output, dec_state,
                                   tar_embedded, W_pgen, b_pgen)

    def scatter_one(a, toks):
        return jax.ops.segment_sum(a.T, toks, num_segments=V).T

    s = jax.vmap(scatter_one)(attn, inp_tokens)

    m, z = _run_stats(s)
    ptr, fin = _run_mix(s, generator_output, p_gen3, m, z)
    return fin, ptr, p_gen3.reshape(B, T), loss.reshape(())


# speedup vs baseline: 1.4310x; 1.4310x over previous
import functools

import jax
import jax.numpy as jnp
from jax import lax
from jax.experimental import pallas as pl
from jax.experimental.pallas import tpu as pltpu
from jax.experimental.pallas import tpu_sc as plsc

B, T, I, H, V, D = 8, 256, 1024, 8, 10000, 512

TT = 128
VT = 2048
NV = (V + VT - 1) // VT
NSUB = 16
VP = 10240
IR = I // NSUB
VR = VP // NSUB
ZR = 128


def _head_kernel(ah_ref, enc_ref, dec_ref, tar_ref, tok_ref, w_ref, b_ref,
                 attnt_ref, pgen_ref, m_ref, z_ref, loss_ref):
    bi = pl.program_id(0)
    tj = pl.program_id(1)
    attn = jnp.mean(ah_ref[0], axis=0)
    attnt_ref[0] = jnp.swapaxes(attn, 0, 1)

    ctx = jnp.dot(attn, enc_ref[0], preferred_element_type=jnp.float32)
    cat = jnp.concatenate([ctx, dec_ref[0], tar_ref[0]], axis=1)
    logits = jnp.dot(cat, w_ref[...], preferred_element_type=jnp.float32)
    pg = jax.nn.sigmoid(logits + b_ref[0, 0])
    pgen_ref[...] = pg.reshape(1, 1, TT)

    tok = tok_ref[0]
    eq = (tok.reshape(I, 1) == tok.reshape(1, I)).astype(jnp.float32)
    g = jnp.dot(attn, eq, preferred_element_type=jnp.float32)
    cnt = jnp.sum(eq, axis=0, keepdims=True)
    recip = 1.0 / cnt
    uniq = jnp.sum(recip)
    m = jnp.max(g, axis=1, keepdims=True)
    zt = jnp.sum(jnp.exp(g - m) * recip, axis=1, keepdims=True)
    z = zt + (V - uniq) * jnp.exp(-m)
    m_ref[...] = m.reshape(1, 1, TT)
    z_ref[...] = z.reshape(1, 1, TT)

    partial = jnp.sum(10.0 * jax.nn.relu(jnp.abs(pg - 0.5) - 0.45))

    @pl.when(jnp.logical_and(bi == 0, tj == 0))
    def _():
        loss_ref[...] = jnp.zeros((1, 1), jnp.float32)

    loss_ref[...] += partial.reshape(1, 1) / (B * T)


def _run_head(attn_heads, enc, dec, tar, tok, w, bvec):
    return pl.pallas_call(
        _head_kernel,
        grid=(B, T // TT),
        in_specs=[
            pl.BlockSpec((1, H, TT, I), lambda b, t: (b, 0, t, 0)),
            pl.BlockSpec((1, I, D), lambda b, t: (b, 0, 0)),
            pl.BlockSpec((1, TT, D), lambda b, t: (b, t, 0)),
            pl.BlockSpec((1, TT, D), lambda b, t: (b, t, 0)),
            pl.BlockSpec((1, 1, I), lambda b, t: (b, 0, 0)),
            pl.BlockSpec((3 * D, 1), lambda b, t: (0, 0)),
            pl.BlockSpec((1, 1), lambda b, t: (0, 0)),
        ],
        out_specs=[
            pl.BlockSpec((1, I, TT), lambda b, t: (b, 0, t)),
            pl.BlockSpec((1, 1, TT), lambda b, t: (b, 0, t)),
            pl.BlockSpec((1, 1, TT), lambda b, t: (b, 0, t)),
            pl.BlockSpec((1, 1, TT), lambda b, t: (b, 0, t)),
            pl.BlockSpec((1, 1), lambda b, t: (0, 0)),
        ],
        out_shape=[
            jax.ShapeDtypeStruct((B, I, T), jnp.float32),
            jax.ShapeDtypeStruct((B, 1, T), jnp.float32),
            jax.ShapeDtypeStruct((B, 1, T), jnp.float32),
            jax.ShapeDtypeStruct((B, 1, T), jnp.float32),
            jax.ShapeDtypeStruct((1, 1), jnp.float32),
        ],
    )(attn_heads, enc, dec, tar, tok.reshape(B, 1, I), w, bvec.reshape(1, 1))


def _sc_scatter_body(attnt_hbm, tok_hbm, s_hbm, table, abuf, zbuf, tbuf):
    c = lax.axis_index("c")
    sid = lax.axis_index("s")

    def _zero_row(r, carry):
        for j in range(TT // 16):
            zbuf[r, pl.ds(j * 16, 16)] = jnp.zeros((16,), jnp.float32)
        return carry

    lax.fori_loop(0, ZR, _zero_row, 0)
    for k in range(VR // ZR):
        pltpu.sync_copy(zbuf, table.at[pl.ds(sid * VR + k * ZR, ZR)])
    plsc.subcore_barrier()

    for b in range(B):
        pltpu.sync_copy(tok_hbm.at[b, pl.ds(sid * IR, IR)], tbuf)
        pltpu.sync_copy(
            attnt_hbm.at[b, pl.ds(sid * IR, IR), pl.ds(c * TT, TT)], abuf)
        pltpu.sync_copy(abuf, table.at[tbuf], add=True)
        plsc.subcore_barrier()
        pltpu.sync_copy(table.at[pl.ds(sid * VR, VR)],
                        s_hbm.at[b, c, pl.ds(sid * VR, VR)])
        plsc.subcore_barrier()
        pltpu.sync_copy(zbuf.at[pl.ds(0, IR)], table.at[tbuf])
        plsc.subcore_barrier()


def _run_scatter(attnt, tok):
    mesh = plsc.VectorSubcoreMesh(core_axis_name="c", subcore_axis_name="s")
    f = pl.kernel(
        _sc_scatter_body,
        out_type=jax.ShapeDtypeStruct((B, 2, VP, TT), jnp.float32),
        mesh=mesh,
        scratch_types=[
            pltpu.VMEM_SHARED((VP, TT), jnp.float32),
            pltpu.VMEM((IR, TT), jnp.float32),
            pltpu.VMEM((ZR, TT), jnp.float32),
            pltpu.VMEM((IR,), jnp.int32),
        ],
    )
    return f(attnt, tok)


def _mix_kernel(s_ref, gen_ref, pg_ref, m_ref, z_ref, ptr_ref, fin_ref):
    m = m_ref[0, 0].reshape(TT, 1)
    zinv = 1.0 / z_ref[0, 0].reshape(TT, 1)
    pg = pg_ref[0, 0].reshape(TT, 1)
    st = jnp.swapaxes(s_ref[0, 0], 0, 1)
    ptr = jnp.exp(st - m) * zinv
    ptr_ref[0] = ptr
    fin_ref[0] = pg * gen_ref[0] + (1.0 - pg) * ptr


def _run_mix(s, gen, pg, m, z):
    return pl.pallas_call(
        _mix_kernel,
        grid=(B, T // TT, NV),
        in_specs=[
            pl.BlockSpec((1, 1, VT, TT), lambda b, t, v: (b, t, v, 0)),
            pl.BlockSpec((1, TT, VT), lambda b, t, v: (b, t, v)),
            pl.BlockSpec((1, 1, TT), lambda b, t, v: (b, 0, t)),
            pl.BlockSpec((1, 1, TT), lambda b, t, v: (b, 0, t)),
            pl.BlockSpec((1, 1, TT), lambda b, t, v: (b, 0, t)),
        ],
        out_specs=[
            pl.BlockSpec((1, TT, VT), lambda b, t, v: (b, t, v)),
            pl.BlockSpec((1, TT, VT), lambda b, t, v: (b, t, v)),
        ],
        out_shape=[
            jax.ShapeDtypeStruct((B, T, V), jnp.float32),
            jax.ShapeDtypeStruct((B, T, V), jnp.float32),
        ],
    )(s, gen, pg, m, z)


def kernel(inp_tokens, tar_embedded, generator_output, enc_output, dec_state,
           attn_heads, W_pgen, b_pgen):
    attnt, p_gen3, m, z, loss = _run_head(
        attn_heads, enc_output, dec_state, tar_embedded, inp_tokens,
        W_pgen, b_pgen)
    s = _run_scatter(attnt, inp_tokens)
    ptr, fin = _run_mix(s, generator_output, p_gen3, m, z)
    return fin, ptr, p_gen3.reshape(B, T), loss.reshape(())
